# Initial kernel scaffold; baseline (speedup 1.0000x reference)
import jax
import jax.numpy as jnp
from jax.experimental import pallas as pl


def kernel(sequence, segment_ids, token_table, segment_table):
    raise NotImplementedError("write your pallas kernel here")


# speedup vs baseline: 1.5014x; 1.5014x over previous
import functools
import math

import jax
import jax.numpy as jnp
from jax import lax
from jax.experimental import pallas as pl
from jax.experimental.pallas import tpu as pltpu
from jax.experimental.pallas import tpu_sc as plsc

NUM_SC_CORES = 2
NUM_SC_SUBCORES = 16
NUM_TILES = NUM_SC_CORES * NUM_SC_SUBCORES

GATHER_CHUNK = 64


def _sc_gather(token_table, flat_idx, n_rows, d):
    rows_per_tile = n_rows // NUM_TILES
    n_chunks = rows_per_tile // GATHER_CHUNK
    mesh = plsc.VectorSubcoreMesh(core_axis_name="c", subcore_axis_name="s")

    @functools.partial(
        pl.kernel,
        out_type=jax.ShapeDtypeStruct((n_rows, d), jnp.float32),
        mesh=mesh,
        scratch_types=[
            pltpu.VMEM((rows_per_tile,), jnp.int32),
            pltpu.VMEM((GATHER_CHUNK, d), jnp.float32),
            pltpu.SemaphoreType.DMA,
        ],
    )
    def sc_kernel(table_hbm, idx_hbm, out_hbm, idx_v, rows_v, sem):
        wid = lax.axis_index("s") * NUM_SC_CORES + lax.axis_index("c")
        base = wid * rows_per_tile
        pltpu.sync_copy(idx_hbm.at[pl.ds(base, rows_per_tile)], idx_v)

        @pl.loop(0, n_chunks)
        def _(c):
            pltpu.async_copy(
                table_hbm.at[idx_v.at[pl.ds(c * GATHER_CHUNK, GATHER_CHUNK)]],
                rows_v,
                sem,
            ).wait()
            pltpu.sync_copy(
                rows_v, out_hbm.at[pl.ds(base + c * GATHER_CHUNK, GATHER_CHUNK)]
            )

    return sc_kernel(token_table, flat_idx)


def _tc_add_body(seg_ids_ref, tok_ref, seg_table_ref, out_ref, pe_ref, *, bs, d, s):
    i = pl.program_id(0)
    b = pl.program_id(1)

    @pl.when(b == 0)
    def _():
        pos = (i * bs + lax.broadcasted_iota(jnp.int32, (bs, d), 0)).astype(
            jnp.float32
        )
        col = lax.broadcasted_iota(jnp.int32, (bs, d), 1).astype(jnp.float32)
        even = 2.0 * jnp.floor(col * 0.5)
        ang = pos * jnp.exp(even * (-math.log(10000.0) / d))
        is_odd = col - even
        pe_ref[...] = jnp.sin(ang + is_odd * (0.5 * math.pi))

    ids = seg_ids_ref[0, 0, :].astype(jnp.float32)[:, None]
    seg = (
        jnp.where(ids == 0.0, 1.0, 0.0) * seg_table_ref[0, :][None, :]
        + jnp.where(ids == 1.0, 1.0, 0.0) * seg_table_ref[1, :][None, :]
        + jnp.where(ids == 2.0, 1.0, 0.0) * seg_table_ref[2, :][None, :]
    )
    out_ref[0] = tok_ref[0] + pe_ref[...] + seg


def _tc_add(tok, seg_ids3, segment_table, bs):
    batch, s, d = tok.shape
    grid = (s // bs, batch)
    return pl.pallas_call(
        functools.partial(_tc_add_body, bs=bs, d=d, s=s),
        grid=grid,
        in_specs=[
            pl.BlockSpec((1, 1, bs), lambda i, b: (b, 0, i)),
            pl.BlockSpec((1, bs, d), lambda i, b: (b, i, 0)),
            pl.BlockSpec((3, d), lambda i, b: (0, 0)),
        ],
        out_specs=pl.BlockSpec((1, bs, d), lambda i, b: (b, i, 0)),
        out_shape=jax.ShapeDtypeStruct((batch, s, d), jnp.float32),
        scratch_shapes=[pltpu.VMEM((bs, d), jnp.float32)],
        compiler_params=pltpu.CompilerParams(
            dimension_semantics=("parallel", "arbitrary")
        ),
    )(seg_ids3, tok, segment_table)


def kernel(sequence, segment_ids, token_table, segment_table):
    batch, s = sequence.shape
    vocab, d = token_table.shape
    n = batch * s
    tok_flat = _sc_gather(token_table, sequence.reshape(n), n, d)
    tok = tok_flat.reshape(batch, s, d)
    return _tc_add(tok, segment_ids.reshape(batch, 1, s), segment_table, bs=512)


# speedup vs baseline: 1.8721x; 1.2469x over previous
import functools
import math

import jax
import jax.numpy as jnp
from jax import lax
from jax.experimental import pallas as pl
from jax.experimental.pallas import tpu as pltpu
from jax.experimental.pallas import tpu_sc as plsc

NUM_SC_CORES = 2
NUM_SC_SUBCORES = 16
NUM_TILES = NUM_SC_CORES * NUM_SC_SUBCORES

GATHER_CHUNK = 64


def _sc_gather(token_table, flat_idx, n_rows, d):
    rows_per_tile = n_rows // NUM_TILES
    n_chunks = rows_per_tile // GATHER_CHUNK
    mesh = plsc.VectorSubcoreMesh(core_axis_name="c", subcore_axis_name="s")

    @functools.partial(
        pl.kernel,
        out_type=jax.ShapeDtypeStruct((n_rows, d), jnp.float32),
        mesh=mesh,
        scratch_types=[
            pltpu.VMEM((rows_per_tile,), jnp.int32),
            pltpu.VMEM((GATHER_CHUNK, d), jnp.float32),
            pltpu.SemaphoreType.DMA,
        ],
    )
    def sc_kernel(table_hbm, idx_hbm, out_hbm, idx_v, rows_v, sem):
        wid = lax.axis_index("s") * NUM_SC_CORES + lax.axis_index("c")
        base = wid * rows_per_tile
        pltpu.sync_copy(idx_hbm.at[pl.ds(base, rows_per_tile)], idx_v)

        @pl.loop(0, n_chunks)
        def _(c):
            pltpu.async_copy(
                table_hbm.at[idx_v.at[pl.ds(c * GATHER_CHUNK, GATHER_CHUNK)]],
                rows_v,
                sem,
            ).wait()
            pltpu.sync_copy(
                rows_v, out_hbm.at[pl.ds(base + c * GATHER_CHUNK, GATHER_CHUNK)]
            )

    return sc_kernel(token_table, flat_idx)


def _tc_add_body(seg_ids_ref, tok_ref, seg_table_ref, out_ref, pe_ref, *, bs, d, s):
    i = pl.program_id(0)
    b = pl.program_id(1)

    @pl.when(b == 0)
    def _():
        nh = bs // 16
        col_h = lax.broadcasted_iota(jnp.int32, (nh, d), 1).astype(jnp.float32)
        theta_h = jnp.exp((2.0 * jnp.floor(col_h * 0.5)) * (-math.log(10000.0) / d))
        s_hi = (
            i * bs + 16 * lax.broadcasted_iota(jnp.int32, (nh, d), 0)
        ).astype(jnp.float32)
        ang_a = s_hi * theta_h
        sin_a = jnp.sin(ang_a).reshape(nh, 1, d)
        cos_a = jnp.sin(ang_a + 0.5 * math.pi).reshape(nh, 1, d)

        col_l = lax.broadcasted_iota(jnp.int32, (1, 16, d), 2).astype(jnp.float32)
        theta_l = jnp.exp((2.0 * jnp.floor(col_l * 0.5)) * (-math.log(10000.0) / d))
        s_lo = lax.broadcasted_iota(jnp.int32, (1, 16, d), 1).astype(jnp.float32)
        ang_b = s_lo * theta_l
        sin_b = jnp.sin(ang_b)
        cos_b = jnp.sin(ang_b + 0.5 * math.pi)

        is_even = (lax.broadcasted_iota(jnp.int32, (1, 16, d), 2) % 2) == 0
        pe3 = jnp.where(
            is_even,
            sin_a * cos_b + cos_a * sin_b,
            cos_a * cos_b - sin_a * sin_b,
        )
        pe_ref[...] = pe3.reshape(bs, d)

    ids = seg_ids_ref[0, 0, :].astype(jnp.float32)[:, None]
    seg = (
        jnp.where(ids == 0.0, 1.0, 0.0) * seg_table_ref[0, :][None, :]
        + jnp.where(ids == 1.0, 1.0, 0.0) * seg_table_ref[1, :][None, :]
        + jnp.where(ids == 2.0, 1.0, 0.0) * seg_table_ref[2, :][None, :]
    )
    out_ref[0] = tok_ref[0] + pe_ref[...] + seg


def _tc_add(tok, seg_ids3, segment_table, bs):
    batch, s, d = tok.shape
    grid = (s // bs, batch)
    return pl.pallas_call(
        functools.partial(_tc_add_body, bs=bs, d=d, s=s),
        grid=grid,
        in_specs=[
            pl.BlockSpec((1, 1, bs), lambda i, b: (b, 0, i)),
            pl.BlockSpec((1, bs, d), lambda i, b: (b, i, 0)),
            pl.BlockSpec((3, d), lambda i, b: (0, 0)),
        ],
        out_specs=pl.BlockSpec((1, bs, d), lambda i, b: (b, i, 0)),
        out_shape=jax.ShapeDtypeStruct((batch, s, d), jnp.float32),
        scratch_shapes=[pltpu.VMEM((bs, d), jnp.float32)],
        compiler_params=pltpu.CompilerParams(
            dimension_semantics=("parallel", "arbitrary")
        ),
    )(seg_ids3, tok, segment_table)


def kernel(sequence, segment_ids, token_table, segment_table):
    batch, s = sequence.shape
    vocab, d = token_table.shape
    n = batch * s
    tok_flat = _sc_gather(token_table, sequence.reshape(n), n, d)
    tok = tok_flat.reshape(batch, s, d)
    return _tc_add(tok, segment_ids.reshape(batch, 1, s), segment_table, bs=512)


# speedup vs baseline: 1.8863x; 1.0076x over previous
import functools
import math

import jax
import jax.numpy as jnp
from jax import lax
from jax.experimental import pallas as pl
from jax.experimental.pallas import tpu as pltpu
from jax.experimental.pallas import tpu_sc as plsc

NUM_SC_CORES = 2
NUM_SC_SUBCORES = 16
NUM_TILES = NUM_SC_CORES * NUM_SC_SUBCORES

GATHER_CHUNK = 64


def _sc_gather(token_table, flat_idx, n_rows, d):
    rows_per_tile = n_rows // NUM_TILES
    n_chunks = rows_per_tile // GATHER_CHUNK
    mesh = plsc.VectorSubcoreMesh(core_axis_name="c", subcore_axis_name="s")

    @functools.partial(
        pl.kernel,
        out_type=jax.ShapeDtypeStruct((n_rows, d), jnp.float32),
        mesh=mesh,
        scratch_types=[
            pltpu.VMEM((rows_per_tile,), jnp.int32),
            pltpu.VMEM((GATHER_CHUNK, d), jnp.float32),
            pltpu.VMEM((GATHER_CHUNK, d), jnp.float32),
            pltpu.SemaphoreType.DMA,
            pltpu.SemaphoreType.DMA,
            pltpu.SemaphoreType.DMA,
            pltpu.SemaphoreType.DMA,
        ],
    )
    def sc_kernel(
        table_hbm, idx_hbm, out_hbm, idx_v, rows0, rows1, gs0, gs1, ws0, ws1
    ):
        wid = lax.axis_index("s") * NUM_SC_CORES + lax.axis_index("c")
        base = wid * rows_per_tile
        pltpu.sync_copy(idx_hbm.at[pl.ds(base, rows_per_tile)], idx_v)

        def g_desc(c, buf, sem):
            return pltpu.make_async_copy(
                table_hbm.at[idx_v.at[pl.ds(c * GATHER_CHUNK, GATHER_CHUNK)]],
                buf,
                sem,
            )

        def w_desc(c, buf, sem):
            return pltpu.make_async_copy(
                buf, out_hbm.at[pl.ds(base + c * GATHER_CHUNK, GATHER_CHUNK)], sem
            )

        g_desc(0, rows0, gs0).start()
        g_desc(1, rows1, gs1).start()

        @pl.loop(0, n_chunks, step=2)
        def _(c):
            g_desc(c, rows0, gs0).wait()
            w_desc(c, rows0, ws0).start()
            g_desc(c + 1, rows1, gs1).wait()
            w_desc(c + 1, rows1, ws1).start()
            w_desc(c, rows0, ws0).wait()

            @pl.when(c + 2 < n_chunks)
            def _():
                g_desc(c + 2, rows0, gs0).start()

            w_desc(c + 1, rows1, ws1).wait()

            @pl.when(c + 3 < n_chunks)
            def _():
                g_desc(c + 3, rows1, gs1).start()

    return sc_kernel(token_table, flat_idx)


def _tc_add_body(seg_ids_ref, tok_ref, seg_table_ref, out_ref, pe_ref, *, bs, d, s):
    i = pl.program_id(0)
    b = pl.program_id(1)

    @pl.when(b == 0)
    def _():
        nh = bs // 16
        col_h = lax.broadcasted_iota(jnp.int32, (nh, d), 1).astype(jnp.float32)
        theta_h = jnp.exp((2.0 * jnp.floor(col_h * 0.5)) * (-math.log(10000.0) / d))
        s_hi = (
            i * bs + 16 * lax.broadcasted_iota(jnp.int32, (nh, d), 0)
        ).astype(jnp.float32)
        ang_a = s_hi * theta_h
        sin_a = jnp.sin(ang_a).reshape(nh, 1, d)
        cos_a = jnp.sin(ang_a + 0.5 * math.pi).reshape(nh, 1, d)

        col_l = lax.broadcasted_iota(jnp.int32, (1, 16, d), 2).astype(jnp.float32)
        theta_l = jnp.exp((2.0 * jnp.floor(col_l * 0.5)) * (-math.log(10000.0) / d))
        s_lo = lax.broadcasted_iota(jnp.int32, (1, 16, d), 1).astype(jnp.float32)
        ang_b = s_lo * theta_l
        sin_b = jnp.sin(ang_b)
        cos_b = jnp.sin(ang_b + 0.5 * math.pi)

        is_even = (lax.broadcasted_iota(jnp.int32, (1, 16, d), 2) % 2) == 0
        pe3 = jnp.where(
            is_even,
            sin_a * cos_b + cos_a * sin_b,
            cos_a * cos_b - sin_a * sin_b,
        )
        pe_ref[...] = pe3.reshape(bs, d)

    ids = seg_ids_ref[0, 0, :].astype(jnp.float32)[:, None]
    seg = (
        jnp.where(ids == 0.0, 1.0, 0.0) * seg_table_ref[0, :][None, :]
        + jnp.where(ids == 1.0, 1.0, 0.0) * seg_table_ref[1, :][None, :]
        + jnp.where(ids == 2.0, 1.0, 0.0) * seg_table_ref[2, :][None, :]
    )
    out_ref[0] = tok_ref[0] + pe_ref[...] + seg


def _tc_add(tok, seg_ids3, segment_table, bs):
    batch, s, d = tok.shape
    grid = (s // bs, batch)
    return pl.pallas_call(
        functools.partial(_tc_add_body, bs=bs, d=d, s=s),
        grid=grid,
        in_specs=[
            pl.BlockSpec((1, 1, bs), lambda i, b: (b, 0, i)),
            pl.BlockSpec((1, bs, d), lambda i, b: (b, i, 0)),
            pl.BlockSpec((3, d), lambda i, b: (0, 0)),
        ],
        out_specs=pl.BlockSpec((1, bs, d), lambda i, b: (b, i, 0)),
        out_shape=jax.ShapeDtypeStruct((batch, s, d), jnp.float32),
        scratch_shapes=[pltpu.VMEM((bs, d), jnp.float32)],
        compiler_params=pltpu.CompilerParams(
            dimension_semantics=("parallel", "arbitrary")
        ),
    )(seg_ids3, tok, segment_table)


def kernel(sequence, segment_ids, token_table, segment_table):
    batch, s = sequence.shape
    vocab, d = token_table.shape
    n = batch * s
    tok_flat = _sc_gather(token_table, sequence.reshape(n), n, d)
    tok = tok_flat.reshape(batch, s, d)
    return _tc_add(tok, segment_ids.reshape(batch, 1, s), segment_table, bs=512)


# speedup vs baseline: 2.0173x; 1.0694x over previous
import functools
import math

import jax
import jax.numpy as jnp
from jax import lax
from jax.experimental import pallas as pl
from jax.experimental.pallas import tpu as pltpu
from jax.experimental.pallas import tpu_sc as plsc

NUM_SC_CORES = 2
NUM_SC_SUBCORES = 16
NUM_TILES = NUM_SC_CORES * NUM_SC_SUBCORES

GATHER_CHUNK = 64


def _sc_gather(token_table, flat_idx, n_rows, d):
    rows_per_tile = n_rows // NUM_TILES
    n_chunks = rows_per_tile // GATHER_CHUNK
    mesh = plsc.VectorSubcoreMesh(core_axis_name="c", subcore_axis_name="s")

    @functools.partial(
        pl.kernel,
        out_type=jax.ShapeDtypeStruct((n_rows, d), jnp.float32),
        mesh=mesh,
        scratch_types=[
            pltpu.VMEM((rows_per_tile,), jnp.int32),
            pltpu.VMEM((GATHER_CHUNK, d), jnp.float32),
            pltpu.VMEM((GATHER_CHUNK, d), jnp.float32),
            pltpu.SemaphoreType.DMA,
            pltpu.SemaphoreType.DMA,
            pltpu.SemaphoreType.DMA,
            pltpu.SemaphoreType.DMA,
        ],
    )
    def sc_kernel(
        table_hbm, idx_hbm, out_hbm, idx_v, rows0, rows1, gs0, gs1, ws0, ws1
    ):
        wid = lax.axis_index("s") * NUM_SC_CORES + lax.axis_index("c")
        base = wid * rows_per_tile
        pltpu.sync_copy(idx_hbm.at[pl.ds(base, rows_per_tile)], idx_v)

        def g_desc(c, buf, sem):
            return pltpu.make_async_copy(
                table_hbm.at[idx_v.at[pl.ds(c * GATHER_CHUNK, GATHER_CHUNK)]],
                buf,
                sem,
            )

        def w_desc(c, buf, sem):
            return pltpu.make_async_copy(
                buf, out_hbm.at[pl.ds(base + c * GATHER_CHUNK, GATHER_CHUNK)], sem
            )

        g_desc(0, rows0, gs0).start()
        g_desc(1, rows1, gs1).start()

        @pl.loop(0, n_chunks, step=2)
        def _(c):
            g_desc(c, rows0, gs0).wait()
            w_desc(c, rows0, ws0).start()
            g_desc(c + 1, rows1, gs1).wait()
            w_desc(c + 1, rows1, ws1).start()
            w_desc(c, rows0, ws0).wait()

            @pl.when(c + 2 < n_chunks)
            def _():
                g_desc(c + 2, rows0, gs0).start()

            w_desc(c + 1, rows1, ws1).wait()

            @pl.when(c + 3 < n_chunks)
            def _():
                g_desc(c + 3, rows1, gs1).start()

    return sc_kernel(token_table, flat_idx)


def _pe_block(i, bs, d):
    nh = bs // 16
    col_h = lax.broadcasted_iota(jnp.int32, (nh, d), 1).astype(jnp.float32)
    theta_h = jnp.exp((2.0 * jnp.floor(col_h * 0.5)) * (-math.log(10000.0) / d))
    s_hi = (i * bs + 16 * lax.broadcasted_iota(jnp.int32, (nh, d), 0)).astype(
        jnp.float32
    )
    ang_a = s_hi * theta_h
    sin_a = jnp.sin(ang_a).reshape(nh, 1, d)
    cos_a = jnp.sin(ang_a + 0.5 * math.pi).reshape(nh, 1, d)

    col_l = lax.broadcasted_iota(jnp.int32, (1, 16, d), 2).astype(jnp.float32)
    theta_l = jnp.exp((2.0 * jnp.floor(col_l * 0.5)) * (-math.log(10000.0) / d))
    s_lo = lax.broadcasted_iota(jnp.int32, (1, 16, d), 1).astype(jnp.float32)
    ang_b = s_lo * theta_l
    sin_b = jnp.sin(ang_b)
    cos_b = jnp.sin(ang_b + 0.5 * math.pi)

    is_even = (lax.broadcasted_iota(jnp.int32, (1, 16, d), 2) % 2) == 0
    pe3 = jnp.where(
        is_even,
        sin_a * cos_b + cos_a * sin_b,
        cos_a * cos_b - sin_a * sin_b,
    )
    return pe3.reshape(bs, d)


def _pe_gen_body(pe_ref, *, bs, d):
    pe_ref[...] = _pe_block(pl.program_id(0), bs, d)


def _pe_gen(s, d, bs):
    return pl.pallas_call(
        functools.partial(_pe_gen_body, bs=bs, d=d),
        grid=(s // bs,),
        in_specs=[],
        out_specs=pl.BlockSpec((bs, d), lambda i: (i, 0)),
        out_shape=jax.ShapeDtypeStruct((s, d), jnp.float32),
        compiler_params=pltpu.CompilerParams(dimension_semantics=("parallel",)),
    )()


def _tc_add_body(seg_ids_ref, tok_ref, pe_ref, seg_table_ref, out_ref):
    ids = seg_ids_ref[0, 0, :][:, None]
    r0 = seg_table_ref[0, :][None, :]
    r1 = seg_table_ref[1, :][None, :]
    r2 = seg_table_ref[2, :][None, :]
    seg = jnp.where(ids == 2, r2, jnp.where(ids == 1, r1, r0))
    out_ref[0] = tok_ref[0] + pe_ref[...] + seg


def _tc_add(tok, seg_ids3, pe, segment_table, bs):
    batch, s, d = tok.shape
    grid = (s // bs, batch)
    return pl.pallas_call(
        _tc_add_body,
        grid=grid,
        in_specs=[
            pl.BlockSpec((1, 1, bs), lambda i, b: (b, 0, i)),
            pl.BlockSpec((1, bs, d), lambda i, b: (b, i, 0)),
            pl.BlockSpec((bs, d), lambda i, b: (i, 0)),
            pl.BlockSpec((3, d), lambda i, b: (0, 0)),
        ],
        out_specs=pl.BlockSpec((1, bs, d), lambda i, b: (b, i, 0)),
        out_shape=jax.ShapeDtypeStruct((batch, s, d), jnp.float32),
        compiler_params=pltpu.CompilerParams(
            dimension_semantics=("parallel", "arbitrary")
        ),
    )(seg_ids3, tok, pe, segment_table)


def kernel(sequence, segment_ids, token_table, segment_table):
    batch, s = sequence.shape
    vocab, d = token_table.shape
    n = batch * s
    pe = _pe_gen(s, d, bs=512)
    tok_flat = _sc_gather(token_table, sequence.reshape(n), n, d)
    tok = tok_flat.reshape(batch, s, d)
    return _tc_add(tok, segment_ids.reshape(batch, 1, s), pe, segment_table, bs=512)
